# baseline (speedup 1.0000x reference)
import functools

import jax
import jax.numpy as jnp
from jax import lax
from jax.experimental import pallas as pl
from jax.experimental.pallas import tpu as pltpu
from jax.experimental.pallas import tpu_sc as plsc

N_TOK = 1024
N_CB = 1024
DIM = 64
TOK_BLK = 1024
CB_BLK = 128


def _argmin_body(x_ref, w_ref, idx_ref):
    xb = x_ref[...]
    run_min = jnp.full((1, TOK_BLK), jnp.inf, dtype=jnp.float32)
    run_idx = jnp.zeros((1, TOK_BLK), dtype=jnp.float32)
    base_iota = lax.broadcasted_iota(
        jnp.int32, (CB_BLK, TOK_BLK), 0).astype(jnp.float32)
    for j in range(N_CB // CB_BLK):
        wc = w_ref[j * CB_BLK:(j + 1) * CB_BLK, :]
        wsq = jnp.sum(wc * wc, axis=1, keepdims=True)
        dots = lax.dot_general(
            wc, xb, (((1,), (1,)), ((), ())),
            preferred_element_type=jnp.float32,
            precision=lax.Precision.HIGHEST,
        )
        scores = wsq - 2.0 * dots
        cmin = jnp.min(scores, axis=0, keepdims=True)
        iota = base_iota + jnp.float32(j * CB_BLK)
        cidx = jnp.min(
            jnp.where(scores == cmin, iota, jnp.float32(2**30)),
            axis=0, keepdims=True)
        upd = cmin < run_min
        run_idx = jnp.where(upd, cidx, run_idx)
        run_min = jnp.where(upd, cmin, run_min)
    idx_ref[...] = run_idx.reshape(TOK_BLK).astype(jnp.int32)


def _tc_argmin(x, weight):
    return pl.pallas_call(
        _argmin_body,
        grid=(N_TOK // TOK_BLK,),
        in_specs=[
            pl.BlockSpec((TOK_BLK, DIM), lambda i: (i, 0)),
            pl.BlockSpec((N_CB, DIM), lambda i: (0, 0)),
        ],
        out_specs=pl.BlockSpec((TOK_BLK,), lambda i: (i,)),
        out_shape=jax.ShapeDtypeStruct((N_TOK,), jnp.int32),
    )(x, weight)


def _sc_gather(table, idx):
    nw = 32
    b_per_w = N_TOK // nw
    mesh = plsc.VectorSubcoreMesh(core_axis_name="c", subcore_axis_name="s")

    h = b_per_w // 2

    @functools.partial(
        pl.kernel,
        mesh=mesh,
        out_type=jax.ShapeDtypeStruct((N_TOK, DIM), jnp.float32),
        scratch_types=[
            pltpu.VMEM((h,), jnp.int32),
            pltpu.VMEM((h,), jnp.int32),
            pltpu.VMEM((h, DIM), jnp.float32),
            pltpu.VMEM((h, DIM), jnp.float32),
            pltpu.SemaphoreType.DMA,
            pltpu.SemaphoreType.DMA,
            pltpu.SemaphoreType.DMA,
        ],
        compiler_params=pltpu.CompilerParams(use_tc_tiling_on_sc=False),
    )
    def k(table_hbm, idx_hbm, out_hbm, idx_a, idx_b, rows_a, rows_b,
          sem_a, sem_b, sem_o):
        wid = lax.axis_index("s") * 2 + lax.axis_index("c")
        base = wid * b_per_w
        pltpu.sync_copy(idx_hbm.at[pl.ds(base, h)], idx_a)
        pltpu.sync_copy(idx_hbm.at[pl.ds(base + h, h)], idx_b)
        ga = pltpu.async_copy(table_hbm.at[idx_a], rows_a, sem_a)
        gb = pltpu.async_copy(table_hbm.at[idx_b], rows_b, sem_b)
        ga.wait()
        oa = pltpu.async_copy(rows_a, out_hbm.at[pl.ds(base, h)], sem_o)
        gb.wait()
        pltpu.sync_copy(rows_b, out_hbm.at[pl.ds(base + h, h)])
        oa.wait()

    return k(table, idx)


def kernel(x, weight):
    q_idx = _tc_argmin(x, weight)
    q_data = _sc_gather(weight, q_idx)
    return (q_data, q_idx)


# speedup vs baseline: 1.0684x; 1.0684x over previous
import functools

import jax
import jax.numpy as jnp
from jax import lax
from jax.experimental import pallas as pl
from jax.experimental.pallas import tpu as pltpu
from jax.experimental.pallas import tpu_sc as plsc

N_TOK = 1024
N_CB = 1024
DIM = 64
TOK_BLK = 1024
CB_BLK = 128


def _argmin_body(x_ref, w_ref, idx_ref):
    xb = x_ref[...]
    run_min = jnp.full((1, TOK_BLK), jnp.inf, dtype=jnp.float32)
    run_idx = jnp.zeros((1, TOK_BLK), dtype=jnp.float32)
    base_iota = lax.broadcasted_iota(
        jnp.int32, (CB_BLK, TOK_BLK), 0).astype(jnp.float32)
    for j in range(N_CB // CB_BLK):
        wc = w_ref[j * CB_BLK:(j + 1) * CB_BLK, :]
        wsq = jnp.sum(wc * wc, axis=1, keepdims=True)
        dots = lax.dot_general(
            wc, xb, (((1,), (1,)), ((), ())),
            preferred_element_type=jnp.float32,
            precision=lax.Precision.HIGHEST,
        )
        scores = wsq - 2.0 * dots
        cmin = jnp.min(scores, axis=0, keepdims=True)
        iota = base_iota + jnp.float32(j * CB_BLK)
        cidx = jnp.min(
            jnp.where(scores == cmin, iota, jnp.float32(2**30)),
            axis=0, keepdims=True)
        upd = cmin < run_min
        run_idx = jnp.where(upd, cidx, run_idx)
        run_min = jnp.where(upd, cmin, run_min)
    idx_ref[...] = run_idx.reshape(TOK_BLK).astype(jnp.int32)


def _tc_argmin(x, weight):
    return pl.pallas_call(
        _argmin_body,
        grid=(N_TOK // TOK_BLK,),
        in_specs=[
            pl.BlockSpec((TOK_BLK, DIM), lambda i: (i, 0)),
            pl.BlockSpec((N_CB, DIM), lambda i: (0, 0)),
        ],
        out_specs=pl.BlockSpec((TOK_BLK,), lambda i: (i,)),
        out_shape=jax.ShapeDtypeStruct((N_TOK,), jnp.int32),
    )(x, weight)


def _sc_gather(table, idx):
    nc = 1
    nw = 16 * nc
    b_per_w = N_TOK // nw
    mesh = plsc.VectorSubcoreMesh(
        core_axis_name="c", subcore_axis_name="s", num_cores=nc)

    @functools.partial(
        pl.kernel,
        mesh=mesh,
        out_type=jax.ShapeDtypeStruct((N_TOK, DIM), jnp.float32),
        scratch_types=[
            pltpu.VMEM((b_per_w,), jnp.int32),
            pltpu.VMEM((b_per_w, DIM), jnp.float32),
            pltpu.SemaphoreType.DMA,
        ],
        compiler_params=pltpu.CompilerParams(use_tc_tiling_on_sc=False),
    )
    def k(table_hbm, idx_hbm, out_hbm, idx_v, rows_v, sem):
        wid = lax.axis_index("s") * nc + lax.axis_index("c")
        base = wid * b_per_w
        pltpu.sync_copy(idx_hbm.at[pl.ds(base, b_per_w)], idx_v)
        pltpu.async_copy(table_hbm.at[idx_v], rows_v, sem).wait()
        pltpu.sync_copy(rows_v, out_hbm.at[pl.ds(base, b_per_w)])

    return k(table, idx)


def kernel(x, weight):
    q_idx = _tc_argmin(x, weight)
    q_data = _sc_gather(weight, q_idx)
    return (q_data, q_idx)
